# Initial kernel scaffold; baseline (speedup 1.0000x reference)
import jax
import jax.numpy as jnp
from jax.experimental import pallas as pl


def kernel(image, class_labels, class_emb, fc_w, fc_b, w2d_0, scale_0, bias_0, w2d_1, scale_1, bias_1, w2d_2, scale_2, bias_2, w2d_3, scale_3, bias_3, w2d_4, scale_4, bias_4):
    raise NotImplementedError("write your pallas kernel here")


# speedup vs baseline: 11.9237x; 11.9237x over previous
import jax
import jax.numpy as jnp
from jax.experimental import pallas as pl
from jax.experimental.pallas import tpu as pltpu

LEAKY_SLOPE = 0.2
VMEM_LIMIT = 64 << 20


def _mm_bn_lrelu_kernel(a_ref, w_ref, scale_ref, bias_ref, o_ref):
    acc = jnp.dot(a_ref[...], w_ref[...], preferred_element_type=jnp.float32)
    y = acc * scale_ref[...] + bias_ref[...]
    o_ref[...] = jnp.where(y >= 0.0, y, LEAKY_SLOPE * y).astype(o_ref.dtype)


def _fit(dim, t):
    t = min(t, dim)
    while t > 1 and dim % t != 0:
        t //= 2
    return max(t, 1)


def _mm_bn_lrelu(a, w, scale, bias, tm):
    M, K = a.shape
    N = w.shape[1]
    tm = _fit(M, tm)
    return pl.pallas_call(
        _mm_bn_lrelu_kernel,
        out_shape=jax.ShapeDtypeStruct((M, N), jnp.bfloat16),
        grid=(M // tm,),
        in_specs=[
            pl.BlockSpec((tm, K), lambda i: (i, 0)),
            pl.BlockSpec((K, N), lambda i: (0, 0)),
            pl.BlockSpec((1, N), lambda i: (0, 0)),
            pl.BlockSpec((1, N), lambda i: (0, 0)),
        ],
        out_specs=pl.BlockSpec((tm, N), lambda i: (i, 0)),
        compiler_params=pltpu.CompilerParams(
            dimension_semantics=("parallel",),
            vmem_limit_bytes=VMEM_LIMIT,
        ),
    )(a, w, scale, bias)


def _im2col(x, k=4, s=2, p=1):
    B, H, W, C = x.shape
    xp = jnp.pad(x, ((0, 0), (p, p), (p, p), (0, 0)))
    Ho = (H + 2 * p - k) // s + 1
    Wo = (W + 2 * p - k) // s + 1
    cols = []
    for ki in range(k):
        for kj in range(k):
            cols.append(xp[:, ki:ki + s * Ho:s, kj:kj + s * Wo:s, :])
    patches = jnp.concatenate(cols, axis=-1)
    return patches.reshape(B * Ho * Wo, k * k * C), Ho, Wo


def _s2d(x):
    B, H, W, C = x.shape
    xp = jnp.pad(x, ((0, 0), (1, 1), (1, 1), (0, 0)))
    Hq, Wq = (H + 2) // 2, (W + 2) // 2
    xq = xp.reshape(B, Hq, 2, Wq, 2, C)
    xq = xq.transpose(0, 1, 3, 2, 4, 5)
    return xq.reshape(B, Hq, Wq, 4 * C)


def _prep_w(w2d, c_in):
    n = w2d.shape[1]
    w = w2d.reshape(2, 2, 2, 2, c_in, n)
    w = w.transpose(0, 2, 1, 3, 4, 5)
    return w.reshape(2, 2, 4 * c_in, n)


def _conv_s2d_kernel(x_ref, w_ref, scale_ref, bias_ref, o_ref, *, Ho, Wo):
    x = x_ref[0]
    c4 = x.shape[-1]
    n = w_ref.shape[3]
    acc = jnp.zeros((Ho * Wo, n), jnp.float32)
    for a in (0, 1):
        for b in (0, 1):
            s = x[a:a + Ho, b:b + Wo, :].reshape(Ho * Wo, c4)
            acc = acc + jnp.dot(s, w_ref[a, b],
                                preferred_element_type=jnp.float32)
    y = acc * scale_ref[...] + bias_ref[...]
    o_ref[0] = jnp.where(y >= 0.0, y, LEAKY_SLOPE * y).astype(o_ref.dtype)


def _conv_s2d(xq, wq, scale, bias, Ho, Wo):
    B, Hq, Wq, C4 = xq.shape
    N = wq.shape[3]
    import functools
    return pl.pallas_call(
        functools.partial(_conv_s2d_kernel, Ho=Ho, Wo=Wo),
        out_shape=jax.ShapeDtypeStruct((B, Ho * Wo, N), jnp.bfloat16),
        grid=(B,),
        in_specs=[
            pl.BlockSpec((1, Hq, Wq, C4), lambda i: (i, 0, 0, 0)),
            pl.BlockSpec((2, 2, C4, N), lambda i: (0, 0, 0, 0)),
            pl.BlockSpec((1, N), lambda i: (0, 0)),
            pl.BlockSpec((1, N), lambda i: (0, 0)),
        ],
        out_specs=pl.BlockSpec((1, Ho * Wo, N), lambda i: (i, 0, 0)),
        compiler_params=pltpu.CompilerParams(
            dimension_semantics=("parallel",),
            vmem_limit_bytes=VMEM_LIMIT,
        ),
    )(xq, wq, scale, bias)


def _conv_fc_kernel(x_ref, w_ref, scale_ref, bias_ref, fcw_ref, fcb_ref,
                    o_ref, *, Ho, Wo):
    x = x_ref[0]
    c4 = x.shape[-1]
    n = w_ref.shape[3]
    acc = jnp.zeros((Ho * Wo, n), jnp.float32)
    for a in (0, 1):
        for b in (0, 1):
            s = x[a:a + Ho, b:b + Wo, :].reshape(Ho * Wo, c4)
            acc = acc + jnp.dot(s, w_ref[a, b],
                                preferred_element_type=jnp.float32)
    y = acc * scale_ref[...] + bias_ref[...]
    y = jnp.where(y >= 0.0, y, LEAKY_SLOPE * y)
    yb = y.astype(jnp.bfloat16).astype(jnp.float32)
    z = jnp.sum(yb * fcw_ref[...]).reshape(1, 1) + fcb_ref[...]
    o_ref[...] = pl.reciprocal(1.0 + jnp.exp(-z), approx=True).reshape(1, 1, 1)


def _conv_fc(xq, wq, scale, bias, fcw, fcb, Ho, Wo):
    B, Hq, Wq, C4 = xq.shape
    N = wq.shape[3]
    import functools
    out = pl.pallas_call(
        functools.partial(_conv_fc_kernel, Ho=Ho, Wo=Wo),
        out_shape=jax.ShapeDtypeStruct((B, 1, 1), jnp.float32),
        grid=(B,),
        in_specs=[
            pl.BlockSpec((1, Hq, Wq, C4), lambda i: (i, 0, 0, 0)),
            pl.BlockSpec((2, 2, C4, N), lambda i: (0, 0, 0, 0)),
            pl.BlockSpec((1, N), lambda i: (0, 0)),
            pl.BlockSpec((1, N), lambda i: (0, 0)),
            pl.BlockSpec((Ho * Wo, N), lambda i: (0, 0)),
            pl.BlockSpec((1, 1), lambda i: (0, 0)),
        ],
        out_specs=pl.BlockSpec((1, 1, 1), lambda i: (i, 0, 0)),
        compiler_params=pltpu.CompilerParams(
            dimension_semantics=("parallel",),
            vmem_limit_bytes=VMEM_LIMIT,
        ),
    )(xq, wq, scale, bias, fcw, fcb)
    return out.reshape(B, 1)


def kernel(image, class_labels, class_emb, fc_w, fc_b,
           w2d_0, scale_0, bias_0, w2d_1, scale_1, bias_1,
           w2d_2, scale_2, bias_2, w2d_3, scale_3, bias_3,
           w2d_4, scale_4, bias_4):
    B, _, H, W = image.shape
    emb = class_emb[class_labels].reshape(B, H, W, 1)
    img = jnp.transpose(image, (0, 2, 3, 1))
    x = jnp.concatenate([img, emb], axis=-1).astype(jnp.bfloat16)

    a1, Ho, Wo = _im2col(x)
    out1 = _mm_bn_lrelu(a1, w2d_0[:, :64], scale_0[:, :64], bias_0[:, :64],
                        tm=8192)
    x = out1.reshape(B, Ho, Wo, 64)

    for w2d, scale, bias, c_in in ((w2d_1, scale_1, bias_1, 64),
                                   (w2d_2, scale_2, bias_2, 128),
                                   (w2d_3, scale_3, bias_3, 256)):
        xq = _s2d(x)
        wq = _prep_w(w2d, c_in)
        Ho, Wo = Ho // 2, Wo // 2
        out = _conv_s2d(xq, wq, scale, bias, Ho, Wo)
        x = out.reshape(B, Ho, Wo, w2d.shape[1])

    xq = _s2d(x)
    wq = _prep_w(w2d_4, 512)
    Ho, Wo = Ho // 2, Wo // 2
    fcw = fc_w.reshape(512, Ho, Wo).transpose(1, 2, 0).reshape(Ho * Wo, 512)
    return _conv_fc(xq, wq, scale_4, bias_4, fcw, fc_b, Ho, Wo)


# speedup vs baseline: 23.1207x; 1.9391x over previous
import functools

import jax
import jax.numpy as jnp
from jax.experimental import pallas as pl
from jax.experimental.pallas import tpu as pltpu

LEAKY_SLOPE = 0.2
VMEM_LIMIT = 64 << 20


def _fold(x):
    B, H, W, C = x.shape
    xp = jnp.pad(x, ((0, 0), (1, 1), (1, 1), (0, 0)))
    return xp.reshape(B, (H + 2) // 2, 2, (W + 2) // 2, 2 * C)


def _taps(x, Ho, Wo):
    M = Ho * Wo
    c2 = x.shape[-1]
    out = []
    for ki in range(4):
        a, r = divmod(ki, 2)
        for b in (0, 1):
            out.append(x[a:a + Ho, r, b:b + Wo, :].reshape(M, c2))
    return out


def _conv_cat_kernel(x_ref, w_ref, scale_ref, bias_ref, o_ref, *, Ho, Wo):
    patches = jnp.concatenate(_taps(x_ref[0], Ho, Wo), axis=-1)
    acc = jnp.dot(patches, w_ref[...], preferred_element_type=jnp.float32)
    y = acc * scale_ref[...] + bias_ref[...]
    o_ref[0] = jnp.where(y >= 0.0, y, LEAKY_SLOPE * y).astype(o_ref.dtype)


def _conv8_kernel(x_ref, w_ref, scale_ref, bias_ref, o_ref, *, Ho, Wo):
    taps = _taps(x_ref[0], Ho, Wo)
    acc = jnp.zeros((Ho * Wo, w_ref.shape[-1]), jnp.float32)
    for i, t in enumerate(taps):
        acc = acc + jnp.dot(t, w_ref[i // 2, i % 2],
                            preferred_element_type=jnp.float32)
    y = acc * scale_ref[...] + bias_ref[...]
    o_ref[0] = jnp.where(y >= 0.0, y, LEAKY_SLOPE * y).astype(o_ref.dtype)


def _conv8_fc_kernel(x_ref, w_ref, scale_ref, bias_ref, fcw_ref, fcb_ref,
                     o_ref, *, Ho, Wo):
    taps = _taps(x_ref[0], Ho, Wo)
    acc = jnp.zeros((Ho * Wo, w_ref.shape[-1]), jnp.float32)
    for i, t in enumerate(taps):
        acc = acc + jnp.dot(t, w_ref[i // 2, i % 2],
                            preferred_element_type=jnp.float32)
    y = acc * scale_ref[...] + bias_ref[...]
    y = jnp.where(y >= 0.0, y, LEAKY_SLOPE * y)
    yb = y.astype(jnp.bfloat16).astype(jnp.float32)
    z = jnp.sum(yb * fcw_ref[...]).reshape(1, 1) + fcb_ref[...]
    o_ref[...] = pl.reciprocal(1.0 + jnp.exp(-z), approx=True).reshape(1, 1, 1)


def _conv(x, w2d, scale, bias, *, cat):
    B, H, W, C = x.shape
    Ho, Wo = H // 2, W // 2
    N = w2d.shape[1]
    xh = _fold(x)
    Hq, Wq = xh.shape[1], xh.shape[3]
    if cat:
        kern = functools.partial(_conv_cat_kernel, Ho=Ho, Wo=Wo)
        w = w2d
        w_spec = pl.BlockSpec((16 * C, N), lambda i: (0, 0))
    else:
        kern = functools.partial(_conv8_kernel, Ho=Ho, Wo=Wo)
        w = w2d.reshape(4, 2, 2 * C, N)
        w_spec = pl.BlockSpec((4, 2, 2 * C, N), lambda i: (0, 0, 0, 0))
    out = pl.pallas_call(
        kern,
        out_shape=jax.ShapeDtypeStruct((B, Ho * Wo, N), jnp.bfloat16),
        grid=(B,),
        in_specs=[
            pl.BlockSpec((1, Hq, 2, Wq, 2 * C), lambda i: (i, 0, 0, 0, 0)),
            w_spec,
            pl.BlockSpec((1, N), lambda i: (0, 0)),
            pl.BlockSpec((1, N), lambda i: (0, 0)),
        ],
        out_specs=pl.BlockSpec((1, Ho * Wo, N), lambda i: (i, 0, 0)),
        compiler_params=pltpu.CompilerParams(
            dimension_semantics=("parallel",),
            vmem_limit_bytes=VMEM_LIMIT,
        ),
    )(xh, w, scale, bias)
    return out.reshape(B, Ho, Wo, N)


def _conv_fc(x, w2d, scale, bias, fc_w, fc_b):
    B, H, W, C = x.shape
    Ho, Wo = H // 2, W // 2
    N = w2d.shape[1]
    xh = _fold(x)
    Hq, Wq = xh.shape[1], xh.shape[3]
    fcw = fc_w.reshape(N, Ho, Wo).transpose(1, 2, 0).reshape(Ho * Wo, N)
    out = pl.pallas_call(
        functools.partial(_conv8_fc_kernel, Ho=Ho, Wo=Wo),
        out_shape=jax.ShapeDtypeStruct((B, 1, 1), jnp.float32),
        grid=(B,),
        in_specs=[
            pl.BlockSpec((1, Hq, 2, Wq, 2 * C), lambda i: (i, 0, 0, 0, 0)),
            pl.BlockSpec((4, 2, 2 * C, N), lambda i: (0, 0, 0, 0)),
            pl.BlockSpec((1, N), lambda i: (0, 0)),
            pl.BlockSpec((1, N), lambda i: (0, 0)),
            pl.BlockSpec((Ho * Wo, N), lambda i: (0, 0)),
            pl.BlockSpec((1, 1), lambda i: (0, 0)),
        ],
        out_specs=pl.BlockSpec((1, 1, 1), lambda i: (i, 0, 0)),
        compiler_params=pltpu.CompilerParams(
            dimension_semantics=("parallel",),
            vmem_limit_bytes=VMEM_LIMIT,
        ),
    )(xh, w2d.reshape(4, 2, 2 * C, N), scale, bias, fcw, fc_b)
    return out.reshape(B, 1)


def kernel(image, class_labels, class_emb, fc_w, fc_b,
           w2d_0, scale_0, bias_0, w2d_1, scale_1, bias_1,
           w2d_2, scale_2, bias_2, w2d_3, scale_3, bias_3,
           w2d_4, scale_4, bias_4):
    B, _, H, W = image.shape
    emb = class_emb[class_labels].reshape(B, H, W, 1)
    img = jnp.transpose(image, (0, 2, 3, 1))
    x = jnp.concatenate([img, emb], axis=-1).astype(jnp.bfloat16)

    x = _conv(x, w2d_0[:, :64], scale_0[:, :64], bias_0[:, :64], cat=True)
    x = _conv(x, w2d_1, scale_1, bias_1, cat=False)
    x = _conv(x, w2d_2, scale_2, bias_2, cat=False)
    x = _conv(x, w2d_3, scale_3, bias_3, cat=False)
    return _conv_fc(x, w2d_4, scale_4, bias_4, fc_w, fc_b)


# speedup vs baseline: 24.4478x; 1.0574x over previous
import functools

import jax
import jax.numpy as jnp
from jax.experimental import pallas as pl
from jax.experimental.pallas import tpu as pltpu

LEAKY_SLOPE = 0.2
VMEM_LIMIT = 64 << 20

_DN = (((2,), (0,)), ((), ()))


def _fold(x):
    B, H, W, C = x.shape
    xp = jnp.pad(x, ((0, 0), (1, 1), (1, 1), (0, 0)))
    return xp.reshape(B, (H + 2) // 2, 2, (W + 2) // 2, 2 * C)


def _tap_acc(x, w_ref, Ho, Wo):
    acc = None
    for ki in range(4):
        a, r = divmod(ki, 2)
        for b in (0, 1):
            t = x[a:a + Ho, r, b:b + Wo, :]
            d = jax.lax.dot_general(t, w_ref[ki, b], _DN,
                                    preferred_element_type=jnp.float32)
            acc = d if acc is None else acc + d
    return acc


def _conv8_kernel(x_ref, w_ref, scale_ref, bias_ref, o_ref, *, Ho, Wo):
    y = _tap_acc(x_ref[0], w_ref, Ho, Wo) * scale_ref[...] + bias_ref[...]
    o_ref[0] = jnp.where(y >= 0.0, y, LEAKY_SLOPE * y).astype(o_ref.dtype)


def _conv_cat_kernel(x_ref, w_ref, scale_ref, bias_ref, o_ref, *, Ho, Wo):
    x = x_ref[0]
    taps = [x[ki // 2:ki // 2 + Ho, ki % 2, b:b + Wo, :]
            for ki in range(4) for b in (0, 1)]
    patches = jnp.concatenate(taps, axis=-1)
    y = jax.lax.dot_general(patches, w_ref[...], _DN,
                            preferred_element_type=jnp.float32)
    y = y * scale_ref[...] + bias_ref[...]
    o_ref[0] = jnp.where(y >= 0.0, y, LEAKY_SLOPE * y).astype(o_ref.dtype)


def _conv8_fc_kernel(x_ref, w_ref, scale_ref, bias_ref, fcw_ref, fcb_ref,
                     o_ref, *, Ho, Wo):
    y = _tap_acc(x_ref[0], w_ref, Ho, Wo) * scale_ref[...] + bias_ref[...]
    y = jnp.where(y >= 0.0, y, LEAKY_SLOPE * y)
    yb = y.astype(jnp.bfloat16).astype(jnp.float32)
    z = jnp.sum(yb * fcw_ref[...]).reshape(1, 1) + fcb_ref[...]
    o_ref[...] = pl.reciprocal(1.0 + jnp.exp(-z), approx=True).reshape(1, 1, 1)


def _conv(x, w2d, scale, bias, *, cat=False):
    B, H, W, C = x.shape
    Ho, Wo = H // 2, W // 2
    N = w2d.shape[1]
    xh = _fold(x)
    Hq, Wq = xh.shape[1], xh.shape[3]
    if cat:
        kern = functools.partial(_conv_cat_kernel, Ho=Ho, Wo=Wo)
        w = w2d
        w_spec = pl.BlockSpec((16 * C, N), lambda i: (0, 0))
    else:
        kern = functools.partial(_conv8_kernel, Ho=Ho, Wo=Wo)
        w = w2d.reshape(4, 2, 2 * C, N)
        w_spec = pl.BlockSpec((4, 2, 2 * C, N), lambda i: (0, 0, 0, 0))
    out = pl.pallas_call(
        kern,
        out_shape=jax.ShapeDtypeStruct((B, Ho, Wo, N), jnp.bfloat16),
        grid=(B,),
        in_specs=[
            pl.BlockSpec((1, Hq, 2, Wq, 2 * C), lambda i: (i, 0, 0, 0, 0)),
            w_spec,
            pl.BlockSpec((1, N), lambda i: (0, 0)),
            pl.BlockSpec((1, N), lambda i: (0, 0)),
        ],
        out_specs=pl.BlockSpec((1, Ho, Wo, N), lambda i: (i, 0, 0, 0)),
        compiler_params=pltpu.CompilerParams(
            dimension_semantics=("arbitrary",),
            vmem_limit_bytes=VMEM_LIMIT,
        ),
    )(xh, w, scale, bias)
    return out


def _conv_fc(x, w2d, scale, bias, fc_w, fc_b):
    B, H, W, C = x.shape
    Ho, Wo = H // 2, W // 2
    N = w2d.shape[1]
    xh = _fold(x)
    Hq, Wq = xh.shape[1], xh.shape[3]
    fcw = fc_w.reshape(N, Ho, Wo).transpose(1, 2, 0)
    out = pl.pallas_call(
        functools.partial(_conv8_fc_kernel, Ho=Ho, Wo=Wo),
        out_shape=jax.ShapeDtypeStruct((B, 1, 1), jnp.float32),
        grid=(B,),
        in_specs=[
            pl.BlockSpec((1, Hq, 2, Wq, 2 * C), lambda i: (i, 0, 0, 0, 0)),
            pl.BlockSpec((4, 2, 2 * C, N), lambda i: (0, 0, 0, 0)),
            pl.BlockSpec((1, N), lambda i: (0, 0)),
            pl.BlockSpec((1, N), lambda i: (0, 0)),
            pl.BlockSpec((Ho, Wo, N), lambda i: (0, 0, 0)),
            pl.BlockSpec((1, 1), lambda i: (0, 0)),
        ],
        out_specs=pl.BlockSpec((1, 1, 1), lambda i: (i, 0, 0)),
        compiler_params=pltpu.CompilerParams(
            dimension_semantics=("arbitrary",),
            vmem_limit_bytes=VMEM_LIMIT,
        ),
    )(xh, w2d.reshape(4, 2, 2 * C, N), scale, bias, fcw, fc_b)
    return out.reshape(B, 1)


def kernel(image, class_labels, class_emb, fc_w, fc_b,
           w2d_0, scale_0, bias_0, w2d_1, scale_1, bias_1,
           w2d_2, scale_2, bias_2, w2d_3, scale_3, bias_3,
           w2d_4, scale_4, bias_4):
    B, _, H, W = image.shape
    emb = class_emb[class_labels].reshape(B, H, W, 1)
    img = jnp.transpose(image, (0, 2, 3, 1))
    x = jnp.concatenate([img, emb], axis=-1).astype(jnp.bfloat16)

    x = _conv(x, w2d_0[:, :64], scale_0[:, :64], bias_0[:, :64], cat=True)
    x = _conv(x, w2d_1, scale_1, bias_1)
    x = _conv(x, w2d_2, scale_2, bias_2)
    x = _conv(x, w2d_3, scale_3, bias_3)
    return _conv_fc(x, w2d_4, scale_4, bias_4, fc_w, fc_b)
